# baseline (speedup 1.0000x reference)
import jax
import jax.numpy as jnp
from jax.experimental import pallas as pl

_B, _N, _D = 8, 2048, 3


def _chamfer_body(a_ref, bt_ref, out_ref):
    b = pl.program_id(0)
    a = a_ref[0]
    bt = bt_ref[0]

    na = jnp.sum(a * a, axis=1, keepdims=True)
    nb = jnp.sum(bt * bt, axis=0, keepdims=True)
    am2 = a * (-2.0)

    q = am2[:, 0:1] * bt[0:1, :] + nb
    q = am2[:, 1:2] * bt[1:2, :] + q
    q = am2[:, 2:3] * bt[2:3, :] + q

    rmin = jnp.min(q, axis=1, keepdims=True)
    cmin = jnp.min(q + na, axis=0, keepdims=True)
    s = (jnp.sum(na + rmin, axis=(0, 1), keepdims=True)
         + jnp.sum(cmin, axis=(0, 1), keepdims=True))

    @pl.when(b == 0)
    def _():
        out_ref[:, :] = jnp.zeros_like(s)

    out_ref[:, :] += s


def kernel(predict_pc, gt_pc):
    gtt = jnp.transpose(gt_pc, (0, 2, 1))
    out = pl.pallas_call(
        _chamfer_body,
        grid=(_B,),
        in_specs=[
            pl.BlockSpec((1, _N, _D), lambda b: (b, 0, 0)),
            pl.BlockSpec((1, _D, _N), lambda b: (b, 0, 0)),
        ],
        out_specs=pl.BlockSpec((1, 1), lambda b: (0, 0)),
        out_shape=jax.ShapeDtypeStruct((1, 1), jnp.float32),
    )(predict_pc, gtt)
    return out[0, 0] / (2.0 * _B * _N)
